# baseline (speedup 1.0000x reference)
import jax
import numpy as np
import jax.numpy as jnp
from jax import lax
from jax.experimental import pallas as pl
from jax.experimental.pallas import tpu as pltpu
from jax.experimental.pallas import tpu_sc as plsc

N_NODES = 10000
NPAD = 10240
E = 320000
D = 128
B = 1024
K = 100
NC = 2
NS = 16
NW = NC * NS
CH = 80
NCHUNK = 128
NSLOT = 2
EPW = NCHUNK * CH
E_PAD = NW * EPW
ROWS_PT = NPAD // NS
PAIRS = B * K
PPW = PAIRS // NW
PC = 128
NPC = PPW // PC

_mesh = plsc.VectorSubcoreMesh(core_axis_name="c", subcore_axis_name="s",
                               num_cores=NC, num_subcores=NS)


def _i32(x):
    return jnp.asarray(x, jnp.int32)


def _worker_id():
    return _i32(lax.axis_index("s")) * _i32(NC) + _i32(lax.axis_index("c"))


def _make_agg(with_deg):
    out_type = [jax.ShapeDtypeStruct((NC, NPAD, D), jnp.float32)]
    scratch = [
        pltpu.VMEM_SHARED((NPAD, D), jnp.float32),
        pltpu.VMEM((EPW,), jnp.int32),
        pltpu.VMEM((NCHUNK, CH), jnp.int32),
    ]
    scratch += [pltpu.VMEM((CH, D), jnp.float32) for _ in range(NSLOT)]
    scratch += [pltpu.SemaphoreType.DMA for _ in range(2 * NSLOT)]
    if with_deg:
        out_type.append(jax.ShapeDtypeStruct((NC, NPAD), jnp.float32))
        scratch += [
            pltpu.VMEM_SHARED((NPAD,), jnp.float32),
            pltpu.VMEM((CH,), jnp.float32),
        ]

    def body(h_hbm, src_hbm, dst3_hbm, z2d_hbm, z1d_hbm, *rest):
        if with_deg:
            part_hbm, degp_hbm = rest[0], rest[1]
            agg_sh, srcidx, dsts = rest[2], rest[3], rest[4]
            rows = rest[5:5 + NSLOT]
            gsem = rest[5 + NSLOT:5 + 2 * NSLOT]
            ssem = rest[5 + 2 * NSLOT:5 + 3 * NSLOT]
            deg_sh, ones = rest[5 + 3 * NSLOT], rest[6 + 3 * NSLOT]
        else:
            part_hbm = rest[0]
            agg_sh, srcidx, dsts = rest[1], rest[2], rest[3]
            rows = rest[4:4 + NSLOT]
            gsem = rest[4 + NSLOT:4 + 2 * NSLOT]
            ssem = rest[4 + 2 * NSLOT:4 + 3 * NSLOT]
        core = lax.axis_index("c")
        sub = lax.axis_index("s")
        wid = _worker_id()
        e0 = wid * _i32(EPW)
        r0 = _i32(sub) * _i32(ROWS_PT)

        pltpu.sync_copy(z2d_hbm, agg_sh.at[pl.ds(r0, ROWS_PT)])
        if with_deg:
            pltpu.sync_copy(z1d_hbm.at[pl.ds(r0, ROWS_PT)],
                            deg_sh.at[pl.ds(r0, ROWS_PT)])
            for j in range(CH // 16):
                ones[pl.ds(j * 16, 16)] = jnp.ones((16,), jnp.float32)
        pltpu.sync_copy(src_hbm.at[pl.ds(e0, EPW)], srcidx)
        pltpu.sync_copy(dst3_hbm.at[wid], dsts)
        plsc.subcore_barrier()

        def _gather(g, k):
            pltpu.async_copy(
                h_hbm.at[srcidx.at[pl.ds(g * _i32(CH), CH)]], rows[k],
                gsem[k])

        def _gwait(k):
            pltpu.make_async_copy(h_hbm.at[pl.ds(0, CH)], rows[k],
                                  gsem[k]).wait()

        def _scat(g, k):
            pltpu.async_copy(rows[k], agg_sh.at[dsts.at[g]], ssem[k],
                             add=True)
            if with_deg:
                pltpu.async_copy(ones, deg_sh.at[dsts.at[g]], ssem[k],
                                 add=True)

        def _swait(k):
            pltpu.make_async_copy(h_hbm.at[pl.ds(0, CH)], rows[k],
                                  ssem[k]).wait()
            if with_deg:
                pltpu.make_async_copy(z1d_hbm.at[pl.ds(0, CH)], ones,
                                      ssem[k]).wait()

        STAG = max(NSLOT // 2, 1)
        for k in range(STAG):
            _gather(_i32(k), k)

        for k in range(NSLOT):
            km = (k + STAG) % NSLOT
            _gwait(k)
            _scat(_i32(k), k)
            if k >= STAG:
                _swait(km)
            _gather(_i32(k + STAG), km)

        @pl.loop(jnp.int32(1), jnp.int32(NCHUNK // NSLOT - 1))
        def chunk_loop(t):
            g = _i32(t) * _i32(NSLOT)
            for k in range(NSLOT):
                km = (k + STAG) % NSLOT
                c = g + _i32(k)
                _gwait(k)
                _scat(c, k)
                _swait(km)
                _gather(c + _i32(STAG), km)

        gl = _i32(NCHUNK - NSLOT)
        for k in range(NSLOT):
            km = (k + STAG) % NSLOT
            _gwait(k)
            _scat(gl + _i32(k), k)
            _swait(km)
            if k < STAG:
                _gather(gl + _i32(k + STAG), km)
        for k in range(STAG, NSLOT):
            _swait(k)
        plsc.subcore_barrier()
        pltpu.sync_copy(agg_sh.at[pl.ds(r0, ROWS_PT)],
                        part_hbm.at[core, pl.ds(r0, ROWS_PT)])
        if with_deg:
            pltpu.sync_copy(deg_sh.at[pl.ds(r0, ROWS_PT)],
                            degp_hbm.at[core, pl.ds(r0, ROWS_PT)])

    return pl.kernel(body, out_type=out_type, mesh=_mesh,
                     scratch_types=scratch)


_agg_deg = _make_agg(True)
_agg = _make_agg(False)


def _z(i):
    return jnp.zeros_like(i)


def _tc_layer(parts, deg_t, w, b):
    R = 1024

    def body(p_ref, d_ref, w_ref, b_ref, o_ref):
        agg = p_ref[0] + p_ref[1]
        deg = d_ref[:, 0:1] + d_ref[:, 1:2]
        x = agg / jnp.maximum(deg, 1.0)
        y = lax.dot_general(x, w_ref[...], (((1,), (1,)), ((), ())),
                            preferred_element_type=jnp.float32)
        o_ref[...] = jnp.tanh(y + b_ref[...])

    return pl.pallas_call(
        body,
        grid=(NPAD // R,),
        in_specs=[
            pl.BlockSpec((NC, R, D), lambda i: (_z(i), i, _z(i))),
            pl.BlockSpec((R, NC), lambda i: (i, _z(i))),
            pl.BlockSpec((D, D), lambda i: (_z(i), _z(i))),
            pl.BlockSpec((1, D), lambda i: (_z(i), _z(i))),
        ],
        out_specs=pl.BlockSpec((R, D), lambda i: (i, _z(i))),
        out_shape=jax.ShapeDtypeStruct((NPAD, D), jnp.float32),
    )(parts, deg_t, w, b.reshape(1, D))


def _score_body(h_hbm, ui_hbm, ii_hbm, out_hbm,
                uall, iall, urows0, irows0, urows1, irows1, tmp, outv,
                semu0, semi0, semu1, semi1):
    p0 = _worker_id() * _i32(PPW)
    pltpu.sync_copy(ui_hbm.at[pl.ds(p0, PPW)], uall)
    pltpu.sync_copy(ii_hbm.at[pl.ds(p0, PPW)], iall)

    def _issue(c, ur, ir, su, si):
        off = c * _i32(PC)
        pltpu.async_copy(h_hbm.at[uall.at[pl.ds(off, PC)]], ur, su)
        pltpu.async_copy(h_hbm.at[iall.at[pl.ds(off, PC)]], ir, si)

    def _wait(ur, ir, su, si):
        pltpu.make_async_copy(h_hbm.at[pl.ds(0, PC)], ur, su).wait()
        pltpu.make_async_copy(h_hbm.at[pl.ds(0, PC)], ir, si).wait()

    def _compute(c, ur, ir):
        base = c * _i32(PC)

        @pl.loop(jnp.int32(0), jnp.int32(PC // 4))
        def pair(q4):
            for k in range(4):
                p = _i32(q4) * _i32(4) + _i32(k)
                acc = ur[p, pl.ds(0, 16)] * ir[p, pl.ds(0, 16)]
                for j in range(1, D // 16):
                    acc = acc + ur[p, pl.ds(j * 16, 16)] * ir[p, pl.ds(j * 16, 16)]
                tmp[pl.ds(p * _i32(16), 16)] = acc

        @pl.loop(jnp.int32(0), jnp.int32(PC // 16))
        def grp(gq):
            g = _i32(gq)
            fbase = (g * _i32(16) + lax.iota(jnp.int32, 16)) * _i32(16)
            res = plsc.load_gather(tmp, [fbase])
            for l in range(1, 16):
                res = res + plsc.load_gather(
                    tmp, [fbase + jnp.full((16,), l, jnp.int32)])
            outv[pl.ds(base + g * _i32(16), 16)] = res

    _issue(_i32(0), urows0, irows0, semu0, semi0)

    @pl.loop(jnp.int32(0), jnp.int32((NPC - 1) // 2))
    def chunk(t):
        g = _i32(t) * _i32(2)
        _issue(g + 1, urows1, irows1, semu1, semi1)
        _wait(urows0, irows0, semu0, semi0)
        _compute(g, urows0, irows0)
        _issue(g + 2, urows0, irows0, semu0, semi0)
        _wait(urows1, irows1, semu1, semi1)
        _compute(g + 1, urows1, irows1)

    _wait(urows0, irows0, semu0, semi0)
    _compute(_i32(NPC - 1), urows0, irows0)

    pltpu.sync_copy(outv, out_hbm.at[pl.ds(p0, PPW)])


_score = pl.kernel(
    _score_body,
    out_type=jax.ShapeDtypeStruct((PAIRS,), jnp.float32),
    mesh=_mesh,
    compiler_params=pltpu.CompilerParams(needs_layout_passes=False),
    scratch_types=[
        pltpu.VMEM((PPW,), jnp.int32),
        pltpu.VMEM((PPW,), jnp.int32),
        pltpu.VMEM((PC, D), jnp.float32),
        pltpu.VMEM((PC, D), jnp.float32),
        pltpu.VMEM((PC, D), jnp.float32),
        pltpu.VMEM((PC, D), jnp.float32),
        pltpu.VMEM((PC * 16,), jnp.float32),
        pltpu.VMEM((PPW,), jnp.float32),
        pltpu.SemaphoreType.DMA,
        pltpu.SemaphoreType.DMA,
        pltpu.SemaphoreType.DMA,
        pltpu.SemaphoreType.DMA,
    ],
)


def kernel(embeddings, W0, b0, W1, b1, edge_index, user_index, item_index):
    h0 = embeddings.astype(jnp.float32)
    npad_e = E_PAD - E
    pad_src = (jnp.arange(npad_e, dtype=jnp.int32) * 37) % N_NODES
    pad_dst = N_NODES + (jnp.arange(npad_e, dtype=jnp.int32) % (NPAD - N_NODES))
    src = jnp.concatenate([edge_index[0].astype(jnp.int32), pad_src])
    dst3 = jnp.concatenate([edge_index[1].astype(jnp.int32), pad_dst]
                           ).reshape(NW, NCHUNK, CH)
    uidx = user_index.reshape(-1).astype(jnp.int32)
    iidx = item_index.reshape(-1).astype(jnp.int32)
    z2d = jnp.zeros((ROWS_PT, D), jnp.float32)
    z1d = jnp.zeros((NPAD,), jnp.float32)

    parts, degparts = _agg_deg(h0, src, dst3, z2d, z1d)
    deg_t = degparts.T
    h1 = _tc_layer(parts, deg_t, W0, b0)
    (parts2,) = _agg(h1, src, dst3, z2d, z1d)
    h2 = _tc_layer(parts2, deg_t, W1, b1)
    return _score(h2, uidx, iidx).reshape(B, K)


# speedup vs baseline: 1.2652x; 1.2652x over previous
import jax
import numpy as np
import jax.numpy as jnp
from jax import lax
from jax.experimental import pallas as pl
from jax.experimental.pallas import tpu as pltpu
from jax.experimental.pallas import tpu_sc as plsc

N_NODES = 10000
NPAD = 10240
E = 320000
D = 128
B = 1024
K = 100
NC = 2
NS = 16
NW = NC * NS
CH = 128
NCHUNK = 80
NPHASE = 2
CPP = NCHUNK // NPHASE
EPP = CPP * CH
EPW = NCHUNK * CH
E_PAD = NW * EPW
ROWS_PT = NPAD // NS
PAIRS = B * K
PPW = PAIRS // NW
PC = 128
NPC = PPW // PC

_mesh = plsc.VectorSubcoreMesh(core_axis_name="c", subcore_axis_name="s",
                               num_cores=NC, num_subcores=NS)


def _i32(x):
    return jnp.asarray(x, jnp.int32)


def _worker_id():
    return _i32(lax.axis_index("s")) * _i32(NC) + _i32(lax.axis_index("c"))


def _make_agg(with_deg):
    out_type = [jax.ShapeDtypeStruct((NC, NPAD, D), jnp.float32)]
    scratch = [
        pltpu.VMEM_SHARED((NPAD, D), jnp.float32),
        pltpu.VMEM((EPP,), jnp.int32),
        pltpu.VMEM((CPP, CH), jnp.int32),
        pltpu.VMEM((CH, D), jnp.float32),
        pltpu.VMEM((CH, D), jnp.float32),
        pltpu.SemaphoreType.DMA,
        pltpu.SemaphoreType.DMA,
    ]
    if with_deg:
        out_type.append(jax.ShapeDtypeStruct((NC, NPAD), jnp.float32))
        scratch += [
            pltpu.VMEM_SHARED((NPAD,), jnp.float32),
            pltpu.VMEM((CH,), jnp.float32),
        ]

    def body(h_hbm, src_hbm, dst4_hbm, z2d_hbm, z1d_hbm, *rest):
        if with_deg:
            (part_hbm, degp_hbm, agg_sh, srcidx, dsts, rows0, rows1,
             sem0, sem1, deg_sh, ones) = rest
        else:
            (part_hbm, agg_sh, srcidx, dsts, rows0, rows1,
             sem0, sem1) = rest
        core = lax.axis_index("c")
        sub = lax.axis_index("s")
        wid = _worker_id()
        e0 = wid * _i32(EPW)
        r0 = _i32(sub) * _i32(ROWS_PT)

        pltpu.sync_copy(z2d_hbm, agg_sh.at[pl.ds(r0, ROWS_PT)])
        if with_deg:
            pltpu.sync_copy(z1d_hbm.at[pl.ds(r0, ROWS_PT)],
                            deg_sh.at[pl.ds(r0, ROWS_PT)])
            for j in range(CH // 16):
                ones[pl.ds(j * 16, 16)] = jnp.ones((16,), jnp.float32)

        def _gather(g, rows, sem):
            pltpu.async_copy(
                h_hbm.at[srcidx.at[pl.ds(g * _i32(CH), CH)]], rows, sem)

        def _gwait(rows, sem):
            pltpu.make_async_copy(h_hbm.at[pl.ds(0, CH)], rows, sem).wait()

        def _scat(g, rows):
            pltpu.sync_copy(rows, agg_sh.at[dsts.at[g]], add=True)
            if with_deg:
                pltpu.sync_copy(ones, deg_sh.at[dsts.at[g]], add=True)

        for p in range(NPHASE):
            pltpu.sync_copy(
                src_hbm.at[pl.ds(e0 + _i32(p * EPP), EPP)], srcidx)
            pltpu.sync_copy(dst4_hbm.at[wid, _i32(p)], dsts)
            if p == 0:
                plsc.subcore_barrier()

            _gather(_i32(0), rows0, sem0)

            @pl.loop(jnp.int32(0), jnp.int32(CPP // 2))
            def chunk_loop(t):
                g = _i32(t) * _i32(2)
                _gather(g + 1, rows1, sem1)
                _gwait(rows0, sem0)
                _scat(g, rows0)

                @pl.when(g + _i32(2) < _i32(CPP))
                def _():
                    _gather(g + 2, rows0, sem0)

                _gwait(rows1, sem1)
                _scat(g + 1, rows1)

        plsc.subcore_barrier()
        pltpu.sync_copy(agg_sh.at[pl.ds(r0, ROWS_PT)],
                        part_hbm.at[core, pl.ds(r0, ROWS_PT)])
        if with_deg:
            pltpu.sync_copy(deg_sh.at[pl.ds(r0, ROWS_PT)],
                            degp_hbm.at[core, pl.ds(r0, ROWS_PT)])

    return pl.kernel(body, out_type=out_type, mesh=_mesh,
                     scratch_types=scratch)


_agg_deg = _make_agg(True)
_agg = _make_agg(False)


def _z(i):
    return jnp.zeros_like(i)


def _tc_layer(parts, deg_t, w, b):
    R = 1024

    def body(p_ref, d_ref, w_ref, b_ref, o_ref):
        agg = p_ref[0] + p_ref[1]
        deg = d_ref[:, 0:1] + d_ref[:, 1:2]
        x = agg / jnp.maximum(deg, 1.0)
        y = lax.dot_general(x, w_ref[...], (((1,), (1,)), ((), ())),
                            preferred_element_type=jnp.float32)
        o_ref[...] = jnp.tanh(y + b_ref[...])

    return pl.pallas_call(
        body,
        grid=(NPAD // R,),
        in_specs=[
            pl.BlockSpec((NC, R, D), lambda i: (_z(i), i, _z(i))),
            pl.BlockSpec((R, NC), lambda i: (i, _z(i))),
            pl.BlockSpec((D, D), lambda i: (_z(i), _z(i))),
            pl.BlockSpec((1, D), lambda i: (_z(i), _z(i))),
        ],
        out_specs=pl.BlockSpec((R, D), lambda i: (i, _z(i))),
        out_shape=jax.ShapeDtypeStruct((NPAD, D), jnp.float32),
    )(parts, deg_t, w, b.reshape(1, D))


def _score_body(h_hbm, ui_hbm, ii_hbm, out_hbm,
                uall, iall, urows0, irows0, urows1, irows1, tmp, outv,
                semu0, semi0, semu1, semi1):
    p0 = _worker_id() * _i32(PPW)
    pltpu.sync_copy(ui_hbm.at[pl.ds(p0, PPW)], uall)
    pltpu.sync_copy(ii_hbm.at[pl.ds(p0, PPW)], iall)

    def _issue(c, ur, ir, su, si):
        off = c * _i32(PC)
        pltpu.async_copy(h_hbm.at[uall.at[pl.ds(off, PC)]], ur, su)
        pltpu.async_copy(h_hbm.at[iall.at[pl.ds(off, PC)]], ir, si)

    def _wait(ur, ir, su, si):
        pltpu.make_async_copy(h_hbm.at[pl.ds(0, PC)], ur, su).wait()
        pltpu.make_async_copy(h_hbm.at[pl.ds(0, PC)], ir, si).wait()

    def _compute(c, ur, ir):
        base = c * _i32(PC)

        @pl.loop(jnp.int32(0), jnp.int32(PC // 4))
        def pair(q4):
            for k in range(4):
                p = _i32(q4) * _i32(4) + _i32(k)
                acc = ur[p, pl.ds(0, 16)] * ir[p, pl.ds(0, 16)]
                for j in range(1, D // 16):
                    acc = acc + ur[p, pl.ds(j * 16, 16)] * ir[p, pl.ds(j * 16, 16)]
                tmp[pl.ds(p * _i32(16), 16)] = acc

        @pl.loop(jnp.int32(0), jnp.int32(PC // 16))
        def grp(gq):
            g = _i32(gq)
            fbase = (g * _i32(16) + lax.iota(jnp.int32, 16)) * _i32(16)
            res = plsc.load_gather(tmp, [fbase])
            for l in range(1, 16):
                res = res + plsc.load_gather(
                    tmp, [fbase + jnp.full((16,), l, jnp.int32)])
            outv[pl.ds(base + g * _i32(16), 16)] = res

    _issue(_i32(0), urows0, irows0, semu0, semi0)

    @pl.loop(jnp.int32(0), jnp.int32((NPC - 1) // 2))
    def chunk(t):
        g = _i32(t) * _i32(2)
        _issue(g + 1, urows1, irows1, semu1, semi1)
        _wait(urows0, irows0, semu0, semi0)
        _compute(g, urows0, irows0)
        _issue(g + 2, urows0, irows0, semu0, semi0)
        _wait(urows1, irows1, semu1, semi1)
        _compute(g + 1, urows1, irows1)

    _wait(urows0, irows0, semu0, semi0)
    _compute(_i32(NPC - 1), urows0, irows0)

    pltpu.sync_copy(outv, out_hbm.at[pl.ds(p0, PPW)])


_score = pl.kernel(
    _score_body,
    out_type=jax.ShapeDtypeStruct((PAIRS,), jnp.float32),
    mesh=_mesh,
    compiler_params=pltpu.CompilerParams(needs_layout_passes=False),
    scratch_types=[
        pltpu.VMEM((PPW,), jnp.int32),
        pltpu.VMEM((PPW,), jnp.int32),
        pltpu.VMEM((PC, D), jnp.float32),
        pltpu.VMEM((PC, D), jnp.float32),
        pltpu.VMEM((PC, D), jnp.float32),
        pltpu.VMEM((PC, D), jnp.float32),
        pltpu.VMEM((PC * 16,), jnp.float32),
        pltpu.VMEM((PPW,), jnp.float32),
        pltpu.SemaphoreType.DMA,
        pltpu.SemaphoreType.DMA,
        pltpu.SemaphoreType.DMA,
        pltpu.SemaphoreType.DMA,
    ],
)


def kernel(embeddings, W0, b0, W1, b1, edge_index, user_index, item_index):
    h0 = embeddings.astype(jnp.float32)
    npad_e = E_PAD - E
    pad_src = (jnp.arange(npad_e, dtype=jnp.int32) * 37) % N_NODES
    pad_dst = N_NODES + (jnp.arange(npad_e, dtype=jnp.int32) % (NPAD - N_NODES))
    src = jnp.concatenate([edge_index[0].astype(jnp.int32), pad_src])
    dst3 = jnp.concatenate([edge_index[1].astype(jnp.int32), pad_dst]
                           ).reshape(NW, NPHASE, CPP, CH)
    uidx = user_index.reshape(-1).astype(jnp.int32)
    iidx = item_index.reshape(-1).astype(jnp.int32)
    z2d = jnp.zeros((ROWS_PT, D), jnp.float32)
    z1d = jnp.zeros((NPAD,), jnp.float32)

    parts, degparts = _agg_deg(h0, src, dst3, z2d, z1d)
    deg_t = degparts.T
    h1 = _tc_layer(parts, deg_t, W0, b0)
    (parts2,) = _agg(h1, src, dst3, z2d, z1d)
    h2 = _tc_layer(parts2, deg_t, W1, b1)
    return _score(h2, uidx, iidx).reshape(B, K)
